# Initial kernel scaffold; baseline (speedup 1.0000x reference)
import jax
import jax.numpy as jnp
from jax.experimental import pallas as pl


def kernel(h, edge_index, W1, b1, W2, b2, eps, ln_scale, ln_bias):
    raise NotImplementedError("write your pallas kernel here")


# speedup vs baseline: 3.3900x; 3.3900x over previous
import functools

import jax
import jax.numpy as jnp
from jax import lax
from jax.experimental import pallas as pl
from jax.experimental.pallas import tpu as pltpu
from jax.experimental.pallas import tpu_sc as plsc

_NC = 2
_NS = 16
_CHUNK = 128


def _sc_aggregate(h_lo, h_hi, src_i, dst_i, zer, *, np_, rpw, npw):
    mesh = plsc.VectorSubcoreMesh(core_axis_name="c", subcore_axis_name="s")
    fdt = jax.ShapeDtypeStruct((np_, 128), jnp.float32)
    rpp = rpw // 2

    @functools.partial(
        pl.kernel,
        out_type=(fdt, fdt),
        mesh=mesh,
        scratch_types=[
            pltpu.VMEM((rpp, _CHUNK), jnp.int32),
            pltpu.VMEM((rpp, _CHUNK), jnp.int32),
            pltpu.VMEM((_CHUNK, 128), jnp.float32),
            pltpu.VMEM((_CHUNK, 128), jnp.float32),
            pltpu.VMEM_SHARED((np_, 128), jnp.float32),
            pltpu.SemaphoreType.DMA,
            pltpu.SemaphoreType.DMA,
        ],
    )
    def agg_kernel(h_lo_hbm, h_hi_hbm, src_hbm, dst_hbm, zer_hbm,
                   out_lo, out_hi, idx_s, idx_d, rows0, rows1, shared,
                   sem0, sem1):
        c = lax.axis_index("c")
        s = lax.axis_index("s")
        nbase = s * npw

        pltpu.sync_copy(zer_hbm, shared.at[pl.ds(nbase, npw)])
        plsc.subcore_barrier()

        def phase(h_ref, p):
            row0 = s * rpw + p * rpp
            pltpu.sync_copy(src_hbm.at[pl.ds(row0, rpp)], idx_s)
            pltpu.sync_copy(dst_hbm.at[pl.ds(row0, rpp)], idx_d)

            pltpu.async_copy(h_ref.at[idx_s.at[0]], rows0, sem0)

            @pl.loop(0, rpp, step=2)
            def _(g):
                pltpu.async_copy(h_ref.at[idx_s.at[g + 1]], rows1, sem1)
                pltpu.make_async_copy(h_ref.at[idx_s.at[g]], rows0, sem0).wait()
                pltpu.sync_copy(rows0, shared.at[idx_d.at[g]], add=True)

                @pl.when(g + 2 < rpp)
                def _():
                    pltpu.async_copy(h_ref.at[idx_s.at[g + 2]], rows0, sem0)

                pltpu.make_async_copy(h_ref.at[idx_s.at[g + 1]], rows1, sem1).wait()
                pltpu.sync_copy(rows1, shared.at[idx_d.at[g + 1]], add=True)

        def run(h_ref):
            phase(h_ref, 0)
            phase(h_ref, 1)

        @pl.when(c == 0)
        def _():
            run(h_lo_hbm)

        @pl.when(c == 1)
        def _():
            run(h_hi_hbm)

        plsc.subcore_barrier()

        @pl.when(c == 0)
        def _():
            pltpu.sync_copy(shared.at[pl.ds(nbase, npw)],
                            out_lo.at[pl.ds(nbase, npw)])

        @pl.when(c == 1)
        def _():
            pltpu.sync_copy(shared.at[pl.ds(nbase, npw)],
                            out_hi.at[pl.ds(nbase, npw)])

    return agg_kernel(h_lo, h_hi, src_i, dst_i, zer)


def _tc_mlp_ln(h, alo, ahi, W1, b1, W2, b2, ope, ls, lb, *, bn):
    n, d = h.shape

    def body(h_ref, alo_ref, ahi_ref, w1_ref, b1_ref, w2_ref, b2_ref,
             ope_ref, ls_ref, lb_ref, o_ref):
        hb = h_ref[...]
        pre = ope_ref[0, 0] * hb + jnp.concatenate(
            [alo_ref[...], ahi_ref[...]], axis=1)
        hid = jnp.maximum(
            jnp.dot(pre, w1_ref[...], precision=lax.Precision.HIGHEST)
            + b1_ref[...], 0.0)
        hc = jnp.dot(hid, w2_ref[...], precision=lax.Precision.HIGHEST) \
            + b2_ref[...] + hb
        mu = jnp.mean(hc, axis=1, keepdims=True)
        xc = hc - mu
        var = jnp.mean(xc * xc, axis=1, keepdims=True)
        out = xc * lax.rsqrt(var + 1e-5) * ls_ref[...] + lb_ref[...]
        o_ref[...] = jnp.maximum(out, 0.0)

    full = lambda shape: pl.BlockSpec(shape, lambda i: (0, 0))
    return pl.pallas_call(
        body,
        grid=(n // bn,),
        in_specs=[
            pl.BlockSpec((bn, d), lambda i: (i, 0)),
            pl.BlockSpec((bn, 128), lambda i: (i, 0)),
            pl.BlockSpec((bn, 128), lambda i: (i, 0)),
            full((d, d)),
            full((1, d)),
            full((d, d)),
            full((1, d)),
            full((1, 1)),
            full((1, d)),
            full((1, d)),
        ],
        out_specs=pl.BlockSpec((bn, d), lambda i: (i, 0)),
        out_shape=jax.ShapeDtypeStruct((n, d), jnp.float32),
    )(h, alo, ahi, W1, b1, W2, b2, ope, ls, lb)


def kernel(h, edge_index, W1, b1, W2, b2, eps, ln_scale, ln_bias):
    n, d = h.shape
    e = edge_index.shape[1]
    assert d == 256

    npw = -(-n // _NS)
    npw = -(-npw // 8) * 8
    np_ = npw * _NS
    per = _NS * _CHUNK * 16
    e_pad = -(-e // per) * per
    r = e_pad // _CHUNK
    rpw = r // _NS

    src = edge_index[0].astype(jnp.int32)
    dst = edge_index[1].astype(jnp.int32)
    pad = jnp.full((e_pad - e,), n, jnp.int32)
    src_i = jnp.concatenate([src, pad]).reshape(r, _CHUNK)
    dst_i = jnp.concatenate([dst, pad]).reshape(r, _CHUNK)

    h_lo = jnp.pad(h[:, :128], ((0, np_ - n), (0, 0)))
    h_hi = jnp.pad(h[:, 128:], ((0, np_ - n), (0, 0)))
    zer = jnp.zeros((npw, 128), jnp.float32)

    alo, ahi = _sc_aggregate(h_lo, h_hi, src_i, dst_i, zer,
                             np_=np_, rpw=rpw, npw=npw)

    ope = (1.0 + eps).astype(jnp.float32).reshape(1, 1)
    return _tc_mlp_ln(h, alo, ahi, W1,
                      b1.reshape(1, d), W2, b2.reshape(1, d),
                      ope, ln_scale.reshape(1, d), ln_bias.reshape(1, d),
                      bn=1000)
